# baseline (speedup 1.0000x reference)
import functools

import jax
import jax.numpy as jnp
from jax.experimental import pallas as pl

_LOG2E = 1.4426950408889634


def _body(J, x_ref, m_ref, ws_ref, W1e_ref, W2_ref, b2_ref, out_ref):
    m = m_ref[...]
    acc0 = jnp.dot(m * ws_ref[...], b2_ref[...], preferred_element_type=jnp.float32)
    acc1 = jnp.zeros_like(acc0)
    x = x_ref[...]
    for j in range(J):
        h = jnp.dot(x, W1e_ref[j], preferred_element_type=jnp.float32)
        hh = h * jnp.float32(0.5)
        t = jnp.tanh(hh)
        a = (hh * t + hh).astype(jnp.bfloat16)
        d = jnp.dot(a, W2_ref[j], preferred_element_type=jnp.float32)
        if j % 2 == 0:
            acc0 = acc0 + d
        else:
            acc1 = acc1 + d
    out_ref[...] = acc0 + acc1


def kernel(input, W1, b1, W2, b2, ws, target_joint_mask, target_heading):
    B, J, K = input.shape
    D = b1.shape[1]
    BB = 512
    KE = J * K + J
    mask_f = jnp.concatenate(
        [target_joint_mask, target_heading[:, None]], axis=1
    ).astype(jnp.float32)
    ws2d = ws.reshape(1, J)
    x96 = jnp.concatenate(
        [input.reshape(B, J * K) * jnp.repeat(mask_f, K, axis=1), mask_f], axis=1
    ).astype(jnp.bfloat16)
    jj = jnp.arange(J)
    W1e = (
        jnp.zeros((J, KE, D), jnp.float32)
        .at[jj[:, None], (K * jj)[:, None] + jnp.arange(K)[None, :], :]
        .set(W1)
        .at[jj, J * K + jj, :]
        .set(b1)
        .astype(jnp.bfloat16)
    )
    W2s = (W2 * ws[:, None, None]).astype(jnp.bfloat16)

    body = functools.partial(_body, J)
    out = pl.pallas_call(
        body,
        grid=(B // BB,),
        in_specs=[
            pl.BlockSpec((BB, KE), lambda i: (i, 0)),
            pl.BlockSpec((BB, J), lambda i: (i, 0)),
            pl.BlockSpec((1, J), lambda i: (0, 0)),
            pl.BlockSpec((J, KE, D), lambda i: (0, 0, 0)),
            pl.BlockSpec((J, D, D), lambda i: (0, 0, 0)),
            pl.BlockSpec((J, D), lambda i: (0, 0)),
        ],
        out_specs=pl.BlockSpec((BB, D), lambda i: (i, 0)),
        out_shape=jax.ShapeDtypeStruct((B, D), jnp.float32),
    )(x96, mask_f, ws2d, W1e, W2s, b2)
    return out


# speedup vs baseline: 1.0088x; 1.0088x over previous
import functools

import jax
import jax.numpy as jnp
from jax.experimental import pallas as pl

_LOG2E = 1.4426950408889634


def _body(J, x_ref, m_ref, ws_ref, W1e_ref, W2_ref, b2_ref, out_ref):
    m = m_ref[...]
    acc0 = jnp.dot(m * ws_ref[...], b2_ref[...], preferred_element_type=jnp.float32)
    acc1 = jnp.zeros_like(acc0)
    x = x_ref[...]
    for j in range(J):
        hh = jnp.dot(x, W1e_ref[j], preferred_element_type=jnp.float32)
        t = jnp.tanh(hh)
        a = (hh * t + hh).astype(jnp.bfloat16)
        d = jnp.dot(a, W2_ref[j], preferred_element_type=jnp.float32)
        if j % 2 == 0:
            acc0 = acc0 + d
        else:
            acc1 = acc1 + d
    out_ref[...] = acc0 + acc1


def kernel(input, W1, b1, W2, b2, ws, target_joint_mask, target_heading):
    B, J, K = input.shape
    D = b1.shape[1]
    BB = 512
    KE = J * K + J
    mask_f = jnp.concatenate(
        [target_joint_mask, target_heading[:, None]], axis=1
    ).astype(jnp.float32)
    ws2d = ws.reshape(1, J)
    x96 = jnp.concatenate(
        [input.reshape(B, J * K) * jnp.repeat(mask_f, K, axis=1), mask_f], axis=1
    ).astype(jnp.bfloat16)
    jj = jnp.arange(J)
    W1e = (
        jnp.zeros((J, KE, D), jnp.float32)
        .at[jj[:, None], (K * jj)[:, None] + jnp.arange(K)[None, :], :]
        .set(W1)
        .at[jj, J * K + jj, :]
        .set(b1)
        .astype(jnp.bfloat16)
    ) * jnp.bfloat16(0.5)
    W2s = (W2 * ws[:, None, None]).astype(jnp.bfloat16)

    body = functools.partial(_body, J)
    out = pl.pallas_call(
        body,
        grid=(B // BB,),
        in_specs=[
            pl.BlockSpec((BB, KE), lambda i: (i, 0)),
            pl.BlockSpec((BB, J), lambda i: (i, 0)),
            pl.BlockSpec((1, J), lambda i: (0, 0)),
            pl.BlockSpec((J, KE, D), lambda i: (0, 0, 0)),
            pl.BlockSpec((J, D, D), lambda i: (0, 0, 0)),
            pl.BlockSpec((J, D), lambda i: (0, 0)),
        ],
        out_specs=pl.BlockSpec((BB, D), lambda i: (i, 0)),
        out_shape=jax.ShapeDtypeStruct((B, D), jnp.float32),
    )(x96, mask_f, ws2d, W1e, W2s, b2)
    return out


# speedup vs baseline: 1.0164x; 1.0075x over previous
import functools

import jax
import jax.numpy as jnp
from jax.experimental import pallas as pl

_LOG2E = 1.4426950408889634


def _body(J, x_ref, m_ref, ws_ref, W1e_ref, W2_ref, b2_ref, out_ref):
    m = m_ref[...]
    acc0 = jnp.dot(m * ws_ref[...], b2_ref[...], preferred_element_type=jnp.float32)
    acc1 = jnp.zeros_like(acc0)
    x = x_ref[...]
    for j in range(J):
        hh = jnp.dot(x, W1e_ref[j], preferred_element_type=jnp.float32).astype(
            jnp.bfloat16
        )
        t = jnp.tanh(hh)
        a = hh * t + hh
        d = jnp.dot(a, W2_ref[j], preferred_element_type=jnp.float32)
        if j % 2 == 0:
            acc0 = acc0 + d
        else:
            acc1 = acc1 + d
    out_ref[...] = acc0 + acc1


def kernel(input, W1, b1, W2, b2, ws, target_joint_mask, target_heading):
    B, J, K = input.shape
    D = b1.shape[1]
    BB = 512
    KE = J * K + J
    mask_f = jnp.concatenate(
        [target_joint_mask, target_heading[:, None]], axis=1
    ).astype(jnp.float32)
    ws2d = ws.reshape(1, J)
    x96 = jnp.concatenate(
        [input.reshape(B, J * K) * jnp.repeat(mask_f, K, axis=1), mask_f], axis=1
    ).astype(jnp.bfloat16)
    jj = jnp.arange(J)
    W1e = (
        jnp.zeros((J, KE, D), jnp.float32)
        .at[jj[:, None], (K * jj)[:, None] + jnp.arange(K)[None, :], :]
        .set(W1)
        .at[jj, J * K + jj, :]
        .set(b1)
        .astype(jnp.bfloat16)
    ) * jnp.bfloat16(0.5)
    W2s = (W2 * ws[:, None, None]).astype(jnp.bfloat16)

    body = functools.partial(_body, J)
    out = pl.pallas_call(
        body,
        grid=(B // BB,),
        in_specs=[
            pl.BlockSpec((BB, KE), lambda i: (i, 0)),
            pl.BlockSpec((BB, J), lambda i: (i, 0)),
            pl.BlockSpec((1, J), lambda i: (0, 0)),
            pl.BlockSpec((J, KE, D), lambda i: (0, 0, 0)),
            pl.BlockSpec((J, D, D), lambda i: (0, 0, 0)),
            pl.BlockSpec((J, D), lambda i: (0, 0)),
        ],
        out_specs=pl.BlockSpec((BB, D), lambda i: (i, 0)),
        out_shape=jax.ShapeDtypeStruct((B, D), jnp.float32),
    )(x96, mask_f, ws2d, W1e, W2s, b2)
    return out
